# baseline (speedup 1.0000x reference)
import functools

import jax
import jax.numpy as jnp
from jax import lax
from jax.experimental import pallas as pl
from jax.experimental.pallas import tpu as pltpu
from jax.experimental.pallas import tpu_sc as plsc


def _gelu_exact(x):
    return 0.5 * x * (1.0 + jax.lax.erf(x * 0.7071067811865476))


def _router_body(x_ref, gw_ref, gates_ref, sel_ref, *, n_experts):
    xb = x_ref[...]
    z = jax.lax.dot_general(
        xb, gw_ref[...], (((1,), (1,)), ((), ())),
        preferred_element_type=jnp.float32)
    idx = jax.lax.broadcasted_iota(jnp.int32, z.shape, 1)
    m1 = jnp.max(z, axis=1, keepdims=True)
    i1 = jnp.min(jnp.where(z == m1, idx, n_experts), axis=1, keepdims=True)
    sel1 = idx == i1
    zm = jnp.where(sel1, -jnp.inf, z)
    m2 = jnp.max(zm, axis=1, keepdims=True)
    i2 = jnp.min(jnp.where(zm == m2, idx, n_experts), axis=1, keepdims=True)
    sel2 = idx == i2
    sel = sel1 | sel2
    num = jnp.exp(z - m1)
    denom = 1.0 + jnp.exp(m2 - m1)
    gates_ref[...] = jnp.where(sel, num, 0.0) / denom
    sel_ref[...] = sel.astype(jnp.float32)


def _router(x, gate_w, *, block_t, interpret=False):
    n, d = x.shape
    n_experts = gate_w.shape[0]
    n_t = n // block_t
    return pl.pallas_call(
        functools.partial(_router_body, n_experts=n_experts),
        grid=(n_t,),
        in_specs=[
            pl.BlockSpec((block_t, d), lambda t: (t, 0)),
            pl.BlockSpec((n_experts, d), lambda t: (0, 0)),
        ],
        out_specs=[
            pl.BlockSpec((block_t, n_experts), lambda t: (t, 0)),
            pl.BlockSpec((block_t, n_experts), lambda t: (t, 0)),
        ],
        out_shape=[
            jax.ShapeDtypeStruct((n, n_experts), jnp.float32),
            jax.ShapeDtypeStruct((n, n_experts), jnp.float32),
        ],
        interpret=interpret,
    )(x, gate_w)


def _dispatch_plan(selmask, gates, blk, m_max):
    n, e = selmask.shape
    mask = selmask > 0.5
    maski = mask.astype(jnp.int32)
    cnt = jnp.sum(maski, axis=0)
    pcnt = ((cnt + blk - 1) // blk) * blk
    ends = jnp.cumsum(pcnt)
    offs = ends - pcnt
    colcum = jnp.cumsum(maski, axis=0) - maski
    pos = colcum + offs[None, :]
    flat_pos = jnp.where(mask, pos, m_max).ravel()
    tok = jnp.broadcast_to(
        jnp.arange(n, dtype=jnp.int32)[:, None], (n, e)).ravel()
    s_tok = jnp.zeros((m_max,), jnp.int32).at[flat_pos].set(tok, mode="drop")
    s_gate = jnp.zeros((m_max, 1), jnp.float32).at[flat_pos, 0].set(
        gates.ravel(), mode="drop")
    nblk = m_max // blk
    starts = jnp.arange(nblk, dtype=jnp.int32) * blk
    blk_e = jnp.sum((starts[:, None] >= ends[None, :]).astype(jnp.int32),
                    axis=1)
    pa = jnp.min(jnp.where(mask, pos, m_max), axis=1).astype(jnp.int32)
    pb = jnp.max(jnp.where(mask, pos, -1), axis=1).astype(jnp.int32)
    return s_tok, s_gate, blk_e.astype(jnp.int32), pa, pb


def _sc_gather(x, s_tok):
    m = s_tok.shape[0]
    _, d = x.shape
    info = plsc.get_sparse_core_info()
    nc, ns = info.num_cores, info.num_subcores
    nw = nc * ns
    per_w = m // nw
    chunk = 48
    n_chunks = per_w // chunk
    mesh = plsc.VectorSubcoreMesh(core_axis_name="c", subcore_axis_name="s")

    @functools.partial(
        pl.kernel, mesh=mesh,
        out_type=jax.ShapeDtypeStruct((m, d), jnp.float32),
        scratch_types=[
            pltpu.VMEM((per_w,), jnp.int32),
            pltpu.VMEM((chunk, d), jnp.float32),
            pltpu.VMEM((chunk, d), jnp.float32),
            pltpu.SemaphoreType.DMA,
            pltpu.SemaphoreType.DMA,
            pltpu.SemaphoreType.DMA,
            pltpu.SemaphoreType.DMA,
        ],
    )
    def k(x_hbm, tok_hbm, out_hbm, idx_v, r0_v, r1_v, g0, g1, s0, s1):
        wid = lax.axis_index("s") * nc + lax.axis_index("c")
        base = wid * per_w
        pltpu.sync_copy(tok_hbm.at[pl.ds(base, per_w)], idx_v)
        bufs = (r0_v, r1_v)
        gsem = (g0, g1)
        ssem = (s0, s1)
        gh = [None, None]
        sh = [None, None]
        gh[0] = pltpu.async_copy(
            x_hbm.at[idx_v.at[pl.ds(0, chunk)]], bufs[0], gsem[0])
        for c in range(n_chunks):
            cur = c % 2
            nxt = 1 - cur
            gh[cur].wait()
            if c + 1 < n_chunks:
                if sh[nxt] is not None:
                    sh[nxt].wait()
                gh[nxt] = pltpu.async_copy(
                    x_hbm.at[idx_v.at[pl.ds((c + 1) * chunk, chunk)]],
                    bufs[nxt], gsem[nxt])
            sh[cur] = pltpu.async_copy(
                bufs[cur], out_hbm.at[pl.ds(base + c * chunk, chunk)],
                ssem[cur])
        sh[(n_chunks - 1) % 2].wait()
        if sh[n_chunks % 2] is not None:
            sh[n_chunks % 2].wait()

    return k(x, s_tok)


def _sc_combine(yg, pab):
    n2 = pab.shape[0]
    n = n2 // 2
    d = yg.shape[1]
    info = plsc.get_sparse_core_info()
    nc, ns = info.num_cores, info.num_subcores
    nw = nc * ns
    per_w = n // nw
    chunk = 16
    n_chunks = per_w // chunk
    mesh = plsc.VectorSubcoreMesh(core_axis_name="c", subcore_axis_name="s")

    @functools.partial(
        pl.kernel, mesh=mesh,
        out_type=jax.ShapeDtypeStruct((n, d), jnp.float32),
        scratch_types=[
            pltpu.VMEM((2 * per_w,), jnp.int32),
            pltpu.VMEM((2 * chunk, d), jnp.float32),
            pltpu.VMEM((2 * chunk, d), jnp.float32),
            pltpu.SemaphoreType.DMA,
            pltpu.SemaphoreType.DMA,
            pltpu.SemaphoreType.DMA,
            pltpu.SemaphoreType.DMA,
        ],
    )
    def k(yg_hbm, pab_hbm, out_hbm, idx_v, r0_v, r1_v, g0, g1, s0, s1):
        wid = lax.axis_index("s") * nc + lax.axis_index("c")
        base = wid * per_w
        pltpu.sync_copy(pab_hbm.at[pl.ds(2 * base, 2 * per_w)], idx_v)
        bufs = (r0_v, r1_v)
        gsem = (g0, g1)
        ssem = (s0, s1)
        gh = [None, None]
        sh = [None, None]
        gh[0] = pltpu.async_copy(
            yg_hbm.at[idx_v.at[pl.ds(0, 2 * chunk)]], bufs[0], gsem[0])
        for c in range(n_chunks):
            cur = c % 2
            nxt = 1 - cur
            gh[cur].wait()
            if c + 1 < n_chunks:
                if sh[nxt] is not None:
                    sh[nxt].wait()
                gh[nxt] = pltpu.async_copy(
                    yg_hbm.at[idx_v.at[pl.ds((c + 1) * 2 * chunk, 2 * chunk)]],
                    bufs[nxt], gsem[nxt])
            buf = bufs[cur]

            def row_add(r, carry):
                for j in range(d // 16):
                    sl = pl.ds(j * 16, 16)
                    buf[r, sl] = buf[r, sl] + buf[r + chunk, sl]
                return carry

            lax.fori_loop(0, chunk, row_add, 0)
            sh[cur] = pltpu.async_copy(
                buf.at[pl.ds(0, chunk)],
                out_hbm.at[pl.ds(base + c * chunk, chunk)], ssem[cur])
        sh[(n_chunks - 1) % 2].wait()
        if sh[n_chunks % 2] is not None:
            sh[n_chunks % 2].wait()

    return k(yg, pab)


def _moe_body(be_ref, xg_ref, w1_ref, b1_ref, w2_ref, b2_ref, g_ref, out_ref,
              *, n_experts):
    b = pl.program_id(0)
    be = be_ref[b]

    @pl.when(be < n_experts)
    def _():
        h = jax.lax.dot_general(
            xg_ref[...], w1_ref[0], (((1,), (1,)), ((), ())),
            preferred_element_type=jnp.float32) + b1_ref[0]
        h = _gelu_exact(h)
        y = jax.lax.dot_general(
            h, w2_ref[0], (((1,), (1,)), ((), ())),
            preferred_element_type=jnp.float32) + b2_ref[0]
        out_ref[...] = y * g_ref[...]


def _grouped_mlp(xg, s_gate, blk_e, w1, b1, w2, b2, *, blk, interpret=False):
    m_max, d = xg.shape
    n_experts, h_dim, _ = w1.shape
    nblk = m_max // blk
    b1r = b1.reshape(n_experts, 1, h_dim)
    b2r = b2.reshape(n_experts, 1, h_dim)
    last = n_experts - 1
    grid_spec = pltpu.PrefetchScalarGridSpec(
        num_scalar_prefetch=1,
        grid=(nblk,),
        in_specs=[
            pl.BlockSpec((blk, d), lambda b, be: (b, 0)),
            pl.BlockSpec((1, h_dim, d),
                         lambda b, be: (jnp.minimum(be[b], last), 0, 0)),
            pl.BlockSpec((1, 1, h_dim),
                         lambda b, be: (jnp.minimum(be[b], last), 0, 0)),
            pl.BlockSpec((1, h_dim, h_dim),
                         lambda b, be: (jnp.minimum(be[b], last), 0, 0)),
            pl.BlockSpec((1, 1, h_dim),
                         lambda b, be: (jnp.minimum(be[b], last), 0, 0)),
            pl.BlockSpec((blk, 1), lambda b, be: (b, 0)),
        ],
        out_specs=pl.BlockSpec((blk, d), lambda b, be: (b, 0)),
    )
    return pl.pallas_call(
        functools.partial(_moe_body, n_experts=n_experts),
        grid_spec=grid_spec,
        out_shape=jax.ShapeDtypeStruct((m_max, d), jnp.float32),
        compiler_params=pltpu.CompilerParams(
            dimension_semantics=("arbitrary",)),
        interpret=interpret,
    )(blk_e, xg, w1, b1r, w2, b2r, s_gate)


def _run(x, gate_w, w1, b1, w2, b2, *, interpret=False):
    n, d = x.shape
    n_experts = w1.shape[0]
    block_t = 256 if n % 256 == 0 else n
    blk = block_t
    m_max = 2 * n + n_experts * blk

    gates, selmask = _router(x, gate_w, block_t=block_t, interpret=interpret)
    s_tok, s_gate, blk_e, pa, pb = _dispatch_plan(selmask, gates, blk, m_max)
    pab = jnp.concatenate(
        [pa.reshape(-1, 16), pb.reshape(-1, 16)], axis=1).reshape(-1)
    xg = x[s_tok]
    yg = _grouped_mlp(xg, s_gate, blk_e, w1, b1, w2, b2, blk=blk,
                      interpret=interpret)
    p = pab.reshape(-1, 2, 16)
    return yg[p[:, 0, :].reshape(-1)] + yg[p[:, 1, :].reshape(-1)]


def kernel(hidden_states, gate_w, w1, b1, w2, b2):
    bsz, seq, d = hidden_states.shape
    x = hidden_states.reshape(-1, d)
    out = _run(x, gate_w, w1, b1, w2, b2)
    return out.reshape(bsz, seq, d)


# speedup vs baseline: 2.5166x; 2.5166x over previous
import functools

import jax
import jax.numpy as jnp
from jax.experimental import pallas as pl
from jax.experimental.pallas import tpu as pltpu


def _gelu_exact(x):
    return 0.5 * x * (1.0 + jax.lax.erf(x * 0.7071067811865476))


def _body(x_ref, gw_ref, w1_ref, b1_ref, w2_ref, b2_ref, out_ref, g_scr,
          *, block_t, n_experts, n_t):
    e = pl.program_id(0)
    t = pl.program_id(1)
    xb = x_ref[...]

    @pl.when(e == 0)
    def _():
        z = jax.lax.dot_general(
            xb, gw_ref[...], (((1,), (1,)), ((), ())),
            preferred_element_type=jnp.float32)
        idx = jax.lax.broadcasted_iota(jnp.int32, z.shape, 1)
        m1 = jnp.max(z, axis=1, keepdims=True)
        i1 = jnp.min(jnp.where(z == m1, idx, n_experts), axis=1, keepdims=True)
        sel1 = idx == i1
        zm = jnp.where(sel1, -jnp.inf, z)
        m2 = jnp.max(zm, axis=1, keepdims=True)
        i2 = jnp.min(jnp.where(zm == m2, idx, n_experts), axis=1,
                     keepdims=True)
        sel2 = idx == i2
        num = jnp.exp(z - m1)
        denom = 1.0 + jnp.exp(m2 - m1)
        g_scr[pl.ds(t * block_t, block_t), :] = (
            jnp.where(sel1 | sel2, num, 0.0) / denom)

    h = jax.lax.dot_general(
        xb, w1_ref[0], (((1,), (1,)), ((), ())),
        preferred_element_type=jnp.float32) + b1_ref[0]
    h = _gelu_exact(h)
    y = jax.lax.dot_general(
        h, w2_ref[0], (((1,), (1,)), ((), ())),
        preferred_element_type=jnp.float32) + b2_ref[0]
    gb = g_scr[pl.ds(t * block_t, block_t), :]
    eidx = jax.lax.broadcasted_iota(jnp.int32, gb.shape, 1)
    ge = jnp.sum(jnp.where(eidx == e, gb, 0.0), axis=1, keepdims=True)
    contrib = ge * y
    rows = pl.ds(t * block_t, block_t)

    @pl.when(e == 0)
    def _():
        out_ref[rows, :] = contrib

    @pl.when(e != 0)
    def _():
        out_ref[rows, :] = out_ref[rows, :] + contrib


def _run(x, gate_w, w1, b1, w2, b2, *, interpret=False):
    n, d = x.shape
    n_experts, h_dim, _ = w1.shape
    block_t = 256 if n % 256 == 0 else n
    n_t = n // block_t
    b1r = b1.reshape(n_experts, 1, h_dim)
    b2r = b2.reshape(n_experts, 1, h_dim)
    return pl.pallas_call(
        functools.partial(_body, block_t=block_t, n_experts=n_experts,
                          n_t=n_t),
        grid=(n_experts, n_t),
        in_specs=[
            pl.BlockSpec((block_t, d), lambda e, t: (t, 0)),
            pl.BlockSpec((n_experts, d), lambda e, t: (0, 0)),
            pl.BlockSpec((1, h_dim, d), lambda e, t: (e, 0, 0)),
            pl.BlockSpec((1, 1, h_dim), lambda e, t: (e, 0, 0)),
            pl.BlockSpec((1, h_dim, h_dim), lambda e, t: (e, 0, 0)),
            pl.BlockSpec((1, 1, h_dim), lambda e, t: (e, 0, 0)),
        ],
        out_specs=pl.BlockSpec((n, d), lambda e, t: (0, 0)),
        out_shape=jax.ShapeDtypeStruct((n, d), jnp.float32),
        scratch_shapes=[pltpu.VMEM((n, n_experts), jnp.float32)],
        compiler_params=pltpu.CompilerParams(
            dimension_semantics=("arbitrary", "arbitrary")),
        interpret=interpret,
    )(x, gate_w, w1, b1r, w2, b2r)


def kernel(hidden_states, gate_w, w1, b1, w2, b2):
    bsz, seq, d = hidden_states.shape
    x = hidden_states.reshape(-1, d)
    out = _run(x, gate_w, w1, b1, w2, b2)
    return out.reshape(bsz, seq, d)


# speedup vs baseline: 3.0933x; 1.2292x over previous
import functools

import jax
import jax.numpy as jnp
from jax.experimental import pallas as pl
from jax.experimental.pallas import tpu as pltpu


def _gelu_exact(x):
    return 0.5 * x * (1.0 + jax.lax.erf(x * 0.7071067811865476))


def _body(x_ref, gw_ref, w1_ref, b1_ref, w2_ref, b2_ref, out_ref, g_scr,
          *, block_t, n_experts, n_t):
    e = pl.program_id(0)
    t = pl.program_id(1)
    xb = x_ref[...]

    @pl.when(e == 0)
    def _():
        z = jax.lax.dot_general(
            xb, gw_ref[...], (((1,), (1,)), ((), ())),
            preferred_element_type=jnp.float32)
        idx = jax.lax.broadcasted_iota(jnp.int32, z.shape, 1)
        m1 = jnp.max(z, axis=1, keepdims=True)
        i1 = jnp.min(jnp.where(z == m1, idx, n_experts), axis=1, keepdims=True)
        sel1 = idx == i1
        zm = jnp.where(sel1, -jnp.inf, z)
        m2 = jnp.max(zm, axis=1, keepdims=True)
        i2 = jnp.min(jnp.where(zm == m2, idx, n_experts), axis=1,
                     keepdims=True)
        sel2 = idx == i2
        num = jnp.exp(z - m1)
        denom = 1.0 + jnp.exp(m2 - m1)
        g_scr[pl.ds(t * block_t, block_t), :] = (
            jnp.where(sel1 | sel2, num, 0.0) / denom)

    h = jax.lax.dot_general(
        xb, w1_ref[0], (((1,), (1,)), ((), ())),
        preferred_element_type=jnp.float32) + b1_ref[0]
    h = _gelu_exact(h)
    y = jax.lax.dot_general(
        h, w2_ref[0], (((1,), (1,)), ((), ())),
        preferred_element_type=jnp.float32) + b2_ref[0]
    gb = g_scr[pl.ds(t * block_t, block_t), :]
    eidx = jax.lax.broadcasted_iota(jnp.int32, gb.shape, 1)
    ge = jnp.sum(jnp.where(eidx == e, gb, 0.0), axis=1, keepdims=True)
    contrib = ge * y
    rows = pl.ds(t * block_t, block_t)

    @pl.when(e == 0)
    def _():
        out_ref[rows, :] = contrib

    @pl.when(e != 0)
    def _():
        out_ref[rows, :] = out_ref[rows, :] + contrib


def _run(x, gate_w, w1, b1, w2, b2, *, interpret=False):
    n, d = x.shape
    n_experts, h_dim, _ = w1.shape
    block_t = 512 if n % 512 == 0 else n
    n_t = n // block_t
    b1r = b1.reshape(n_experts, 1, h_dim)
    b2r = b2.reshape(n_experts, 1, h_dim)
    return pl.pallas_call(
        functools.partial(_body, block_t=block_t, n_experts=n_experts,
                          n_t=n_t),
        grid=(n_experts, n_t),
        in_specs=[
            pl.BlockSpec((block_t, d), lambda e, t: (t, 0)),
            pl.BlockSpec((n_experts, d), lambda e, t: (0, 0)),
            pl.BlockSpec((1, h_dim, d), lambda e, t: (e, 0, 0)),
            pl.BlockSpec((1, 1, h_dim), lambda e, t: (e, 0, 0)),
            pl.BlockSpec((1, h_dim, h_dim), lambda e, t: (e, 0, 0)),
            pl.BlockSpec((1, 1, h_dim), lambda e, t: (e, 0, 0)),
        ],
        out_specs=pl.BlockSpec((n, d), lambda e, t: (0, 0)),
        out_shape=jax.ShapeDtypeStruct((n, d), jnp.float32),
        scratch_shapes=[pltpu.VMEM((n, n_experts), jnp.float32)],
        compiler_params=pltpu.CompilerParams(
            dimension_semantics=("arbitrary", "arbitrary")),
        interpret=interpret,
    )(x, gate_w, w1, b1r, w2, b2r)


def kernel(hidden_states, gate_w, w1, b1, w2, b2):
    bsz, seq, d = hidden_states.shape
    x = hidden_states.reshape(-1, d)
    out = _run(x, gate_w, w1, b1, w2, b2)
    return out.reshape(bsz, seq, d)


# speedup vs baseline: 3.3518x; 1.0836x over previous
import functools

import jax
import jax.numpy as jnp
from jax.experimental import pallas as pl
from jax.experimental.pallas import tpu as pltpu


def _gelu_exact(x):
    return 0.5 * x * (1.0 + jax.lax.erf(x * 0.7071067811865476))


def _body(x_ref, gw_ref, w1_ref, b1_ref, w2_ref, b2_ref, out_ref, g_scr,
          *, block_t, n_experts, n_t):
    e = pl.program_id(0)
    t = pl.program_id(1)
    xb = x_ref[...]

    @pl.when(e == 0)
    def _():
        z = jax.lax.dot_general(
            xb, gw_ref[...], (((1,), (1,)), ((), ())),
            preferred_element_type=jnp.float32)
        idx = jax.lax.broadcasted_iota(jnp.int32, z.shape, 1)
        m1 = jnp.max(z, axis=1, keepdims=True)
        i1 = jnp.min(jnp.where(z == m1, idx, n_experts), axis=1, keepdims=True)
        sel1 = idx == i1
        zm = jnp.where(sel1, -jnp.inf, z)
        m2 = jnp.max(zm, axis=1, keepdims=True)
        i2 = jnp.min(jnp.where(zm == m2, idx, n_experts), axis=1,
                     keepdims=True)
        sel2 = idx == i2
        num = jnp.exp(z - m1)
        denom = 1.0 + jnp.exp(m2 - m1)
        g_scr[pl.ds(t * block_t, block_t), :] = (
            jnp.where(sel1 | sel2, num, 0.0) / denom)

    h = jax.lax.dot_general(
        xb, w1_ref[0], (((1,), (1,)), ((), ())),
        preferred_element_type=jnp.float32) + b1_ref[0]
    h = _gelu_exact(h)
    y = jax.lax.dot_general(
        h, w2_ref[0], (((1,), (1,)), ((), ())),
        preferred_element_type=jnp.float32) + b2_ref[0]
    gb = g_scr[pl.ds(t * block_t, block_t), :]
    eidx = jax.lax.broadcasted_iota(jnp.int32, gb.shape, 1)
    ge = jnp.sum(jnp.where(eidx == e, gb, 0.0), axis=1, keepdims=True)
    contrib = ge * y
    rows = pl.ds(t * block_t, block_t)

    @pl.when(e == 0)
    def _():
        out_ref[rows, :] = contrib

    @pl.when(e != 0)
    def _():
        out_ref[rows, :] = out_ref[rows, :] + contrib


def _run(x, gate_w, w1, b1, w2, b2, *, interpret=False):
    n, d = x.shape
    n_experts, h_dim, _ = w1.shape
    block_t = 1024 if n % 1024 == 0 else n
    n_t = n // block_t
    b1r = b1.reshape(n_experts, 1, h_dim)
    b2r = b2.reshape(n_experts, 1, h_dim)
    return pl.pallas_call(
        functools.partial(_body, block_t=block_t, n_experts=n_experts,
                          n_t=n_t),
        grid=(n_experts, n_t),
        in_specs=[
            pl.BlockSpec((block_t, d), lambda e, t: (t, 0)),
            pl.BlockSpec((n_experts, d), lambda e, t: (0, 0)),
            pl.BlockSpec((1, h_dim, d), lambda e, t: (e, 0, 0)),
            pl.BlockSpec((1, 1, h_dim), lambda e, t: (e, 0, 0)),
            pl.BlockSpec((1, h_dim, h_dim), lambda e, t: (e, 0, 0)),
            pl.BlockSpec((1, 1, h_dim), lambda e, t: (e, 0, 0)),
        ],
        out_specs=pl.BlockSpec((n, d), lambda e, t: (0, 0)),
        out_shape=jax.ShapeDtypeStruct((n, d), jnp.float32),
        scratch_shapes=[pltpu.VMEM((n, n_experts), jnp.float32)],
        compiler_params=pltpu.CompilerParams(
            dimension_semantics=("arbitrary", "arbitrary")),
        interpret=interpret,
    )(x, gate_w, w1, b1r, w2, b2r)


def kernel(hidden_states, gate_w, w1, b1, w2, b2):
    bsz, seq, d = hidden_states.shape
    x = hidden_states.reshape(-1, d)
    out = _run(x, gate_w, w1, b1, w2, b2)
    return out.reshape(bsz, seq, d)
